# baseline (speedup 1.0000x reference)
import functools

import jax
import jax.numpy as jnp
from jax import lax
from jax.experimental import pallas as pl
from jax.experimental.pallas import tpu as pltpu
from jax.experimental.pallas import tpu_sc as plsc

_START_ID = 2
_PCNK = 16
_RING = 4
_HRING = 4
_LANES = 16
_LWIN = 80


def _build_emb_kernel(B, T, D, n_workers):
    pos_per_w = T // n_workers
    n_pchunks = pos_per_w // _PCNK
    n_units = n_pchunks * B
    rows_per_w = B * T // n_workers
    mesh = plsc.VectorSubcoreMesh(core_axis_name="c", subcore_axis_name="s")

    scratch = (
        [pltpu.VMEM((B * _LWIN,), jnp.int32)]
        + [pltpu.VMEM((_PCNK,), jnp.int32) for _ in range(_RING)]
        + [pltpu.VMEM((_PCNK, D), jnp.float32) for _ in range(_RING)]
        + [pltpu.VMEM((_PCNK, D), jnp.float32) for _ in range(2)]
        + [pltpu.VMEM((_PCNK, D), jnp.float32) for _ in range(_HRING)]
        + [pltpu.SemaphoreType.DMA]
        + [pltpu.SemaphoreType.DMA for _ in range(_RING)]
        + [pltpu.SemaphoreType.DMA for _ in range(_RING)]
        + [pltpu.SemaphoreType.DMA]
        + [pltpu.SemaphoreType.DMA]
    )

    @functools.partial(
        pl.kernel,
        mesh=mesh,
        out_type=(
            jax.ShapeDtypeStruct((B * T, D), jnp.float32),
            jax.ShapeDtypeStruct((B * T, D), jnp.float32),
        ),
        scratch_types=scratch,
    )
    def emb(lab_hbm, wte_hbm, wpe_hbm, hid_hbm, out_hbm, hidout_hbm,
            buf, *rest):
        idx = rest[0:_RING]
        rows = rest[_RING:2 * _RING]
        wrows = rest[2 * _RING:2 * _RING + 2]
        hbuf = rest[2 * _RING + 2:2 * _RING + 2 + _HRING]
        o = 2 * _RING + 2 + _HRING
        slab = rest[o]
        sg = rest[o + 1:o + 1 + _RING]
        ss = rest[o + 1 + _RING:o + 1 + 2 * _RING]
        shl = rest[o + 1 + 2 * _RING]
        shs = rest[o + 2 + 2 * _RING]
        wid = lax.axis_index("s") * 2 + lax.axis_index("c")
        pos_w0 = wid * pos_per_w
        hid_w0 = wid * rows_per_w
        sw = (pos_w0 == 0).astype(jnp.int32)
        lane = lax.iota(jnp.int32, _LANES)
        pend = {}

        lab_cps = [
            pltpu.async_copy(
                lab_hbm.at[pl.ds(b * T + pos_w0 - 8 + 8 * sw, _LWIN - 8)],
                buf.at[pl.ds(_LWIN * b + 8 * sw, _LWIN - 8)], slab)
            for b in range(B)
        ]
        for cp in lab_cps:
            cp.wait()

        def launch(t):
            st = t % _RING
            c, b = divmod(t, B)
            pos0 = pos_w0 + c * _PCNK
            p = pend.setdefault(t, {})
            v = buf[pl.ds(_LWIN * b + 7 + _PCNK * c, _LANES)]
            if c == 0:
                keep = 1 - (1 - jnp.minimum(lane, 1)) * sw
                v = v * keep + _START_ID * (1 - keep)
            idx[st][pl.ds(0, _LANES)] = v
            p["gat"] = pltpu.async_copy(wte_hbm.at[idx[st]], rows[st], sg[st])
            if b == 0:
                p["wpe"] = pltpu.async_copy(
                    wpe_hbm.at[pl.ds(pos0, _PCNK)], wrows[c % 2], sg[st])

        def hload(t):
            pend.setdefault(t, {})["hld"] = pltpu.async_copy(
                hid_hbm.at[pl.ds(hid_w0 + t * _PCNK, _PCNK)],
                hbuf[t % _HRING], shl)

        def hstore(t):
            pend[t]["hst"] = pltpu.async_copy(
                hbuf[t % _HRING],
                hidout_hbm.at[pl.ds(hid_w0 + t * _PCNK, _PCNK)], shs)

        def finish(t):
            st = t % _RING
            c, b = divmod(t, B)
            pos0 = pos_w0 + c * _PCNK
            pend[t]["gat"].wait()
            if b == 0:
                pend[t]["wpe"].wait()
            wr = wrows[c % 2]

            def add_row(j, carry):
                for k in range(D // _LANES):
                    sl = pl.ds(k * _LANES, _LANES)
                    rows[st][j, sl] = rows[st][j, sl] + wr[j, sl]
                return carry

            lax.fori_loop(0, _PCNK, add_row, 0)
            pend[t]["st"] = pltpu.async_copy(
                rows[st], out_hbm.at[pl.ds(b * T + pos0, _PCNK)], ss[st])

        for t in range(_RING):
            launch(t)
        hload(0)
        hload(1)
        for t in range(n_units):
            if t - 2 >= 0:
                pend[t - 2]["hst"].wait()
            if t + 2 < n_units:
                hload(t + 2)
            pend[t]["hld"].wait()
            hstore(t)
            finish(t)
            if t + _RING < n_units:
                pend[t]["st"].wait()
                launch(t + _RING)
        for t in range(n_units - 2, n_units):
            pend[t]["hst"].wait()
        for t in range(n_units - _RING, n_units):
            pend[t]["st"].wait()

    return emb


def kernel(hidden, labels, wte_table, wpe_table):
    B, T = labels.shape
    D = wte_table.shape[1]
    info = plsc.get_sparse_core_info()
    n_workers = info.num_cores * info.num_subcores
    emb = _build_emb_kernel(B, T, D, n_workers)
    out_flat, hid_out = emb(labels.reshape(B * T), wte_table, wpe_table,
                            hidden.reshape(B * T, D))
    return (hid_out.reshape(B, T, D), out_flat.reshape(B, T, D), labels)


# speedup vs baseline: 1.0243x; 1.0243x over previous
import functools

import jax
import jax.numpy as jnp
from jax import lax
from jax.experimental import pallas as pl
from jax.experimental.pallas import tpu as pltpu
from jax.experimental.pallas import tpu_sc as plsc

_START_ID = 2
_PCNK = 16
_RING = 4
_HRING = 4
_LANES = 16
_LWIN = 80


def _build_emb_kernel(B, T, D, n_workers):
    pos_per_w = T // n_workers
    n_pchunks = pos_per_w // _PCNK
    n_units = n_pchunks * B
    rows_per_w = B * T // n_workers
    mesh = plsc.VectorSubcoreMesh(core_axis_name="c", subcore_axis_name="s")

    scratch = (
        [pltpu.VMEM((B * _LWIN,), jnp.int32)]
        + [pltpu.VMEM((_PCNK,), jnp.int32) for _ in range(_RING)]
        + [pltpu.VMEM((_PCNK, D), jnp.float32) for _ in range(_RING)]
        + [pltpu.VMEM((_PCNK, D), jnp.float32) for _ in range(2)]
        + [pltpu.VMEM((_PCNK, D), jnp.float32) for _ in range(_HRING)]
        + [pltpu.SemaphoreType.DMA]
        + [pltpu.SemaphoreType.DMA for _ in range(_RING)]
        + [pltpu.SemaphoreType.DMA for _ in range(_RING)]
        + [pltpu.SemaphoreType.DMA]
        + [pltpu.SemaphoreType.DMA]
    )

    @functools.partial(
        pl.kernel,
        mesh=mesh,
        out_type=(
            jax.ShapeDtypeStruct((B * T, D), jnp.float32),
            jax.ShapeDtypeStruct((B * T, D), jnp.float32),
        ),
        scratch_types=scratch,
    )
    def emb(lab_hbm, wte_hbm, wpe_hbm, hid_hbm, out_hbm, hidout_hbm,
            buf, *rest):
        idx = rest[0:_RING]
        rows = rest[_RING:2 * _RING]
        wrows = rest[2 * _RING:2 * _RING + 2]
        hbuf = rest[2 * _RING + 2:2 * _RING + 2 + _HRING]
        o = 2 * _RING + 2 + _HRING
        slab = rest[o]
        sg = rest[o + 1:o + 1 + _RING]
        ss = rest[o + 1 + _RING:o + 1 + 2 * _RING]
        shl = rest[o + 1 + 2 * _RING]
        shs = rest[o + 2 + 2 * _RING]
        wid = lax.axis_index("s") * 2 + lax.axis_index("c")
        pos_w0 = wid * pos_per_w
        hid_w0 = wid * rows_per_w
        sw = (pos_w0 == 0).astype(jnp.int32)
        lane = lax.iota(jnp.int32, _LANES)
        pend = {}

        lab_cps = [
            pltpu.async_copy(
                lab_hbm.at[pl.ds(b * T + pos_w0 - 8 + 8 * sw, _LWIN - 8)],
                buf.at[pl.ds(_LWIN * b + 8 * sw, _LWIN - 8)], slab)
            for b in range(B)
        ]
        for cp in lab_cps:
            cp.wait()

        def launch(t):
            st = t % _RING
            c, b = divmod(t, B)
            pos0 = pos_w0 + c * _PCNK
            p = pend.setdefault(t, {})
            v = buf[pl.ds(_LWIN * b + 7 + _PCNK * c, _LANES)]
            if c == 0:
                keep = 1 - (1 - jnp.minimum(lane, 1)) * sw
                v = v * keep + _START_ID * (1 - keep)
            idx[st][pl.ds(0, _LANES)] = v
            p["gat"] = pltpu.async_copy(wte_hbm.at[idx[st]], rows[st], sg[st])
            if b == 0:
                p["wpe"] = pltpu.async_copy(
                    wpe_hbm.at[pl.ds(pos0, _PCNK)], wrows[c % 2], sg[st])

        def hload(t):
            pend.setdefault(t, {})["hld"] = pltpu.async_copy(
                hid_hbm.at[pl.ds(hid_w0 + t * _PCNK, _PCNK)],
                hbuf[t % _HRING], shl)

        def hstore(t):
            pend[t]["hst"] = pltpu.async_copy(
                hbuf[t % _HRING],
                hidout_hbm.at[pl.ds(hid_w0 + t * _PCNK, _PCNK)], shs)

        def finish(t):
            st = t % _RING
            c, b = divmod(t, B)
            pos0 = pos_w0 + c * _PCNK
            pend[t]["gat"].wait()
            if b == 0:
                pend[t]["wpe"].wait()
            wr = wrows[c % 2]

            def add_row(j, carry):
                for k in range(D // _LANES):
                    sl = pl.ds(k * _LANES, _LANES)
                    rows[st][j, sl] = rows[st][j, sl] + wr[j, sl]
                return carry

            lax.fori_loop(0, _PCNK, add_row, 0)
            pend[t]["st"] = pltpu.async_copy(
                rows[st], out_hbm.at[pl.ds(b * T + pos0, _PCNK)], ss[st])

        for t in range(_RING):
            launch(t)
        hload(0)
        hload(1)
        for t in range(n_units):
            if t - 2 >= 0:
                pend[t - 2]["hst"].wait()
            if t + 2 < n_units:
                hload(t + 2)
            finish(t)
            pend[t]["hld"].wait()
            hstore(t)
            if t + _RING < n_units:
                pend[t]["st"].wait()
                launch(t + _RING)
        for t in range(n_units - 2, n_units):
            pend[t]["hst"].wait()
        for t in range(n_units - _RING, n_units):
            pend[t]["st"].wait()

    return emb


def kernel(hidden, labels, wte_table, wpe_table):
    B, T = labels.shape
    D = wte_table.shape[1]
    info = plsc.get_sparse_core_info()
    n_workers = info.num_cores * info.num_subcores
    emb = _build_emb_kernel(B, T, D, n_workers)
    out_flat, hid_out = emb(labels.reshape(B * T), wte_table, wpe_table,
                            hidden.reshape(B * T, D))
    return (hid_out.reshape(B, T, D), out_flat.reshape(B, T, D), labels)
